# baseline (speedup 1.0000x reference)
import functools

import jax
import jax.numpy as jnp
from jax.experimental import pallas as pl
from jax.experimental.pallas import tpu as pltpu

_BN_EPS = 1e-5
_VMEM_LIMIT = 56 << 20


def _stats_kernel(x_ref, w_ref, sum_ref, ssq_ref, *, nb):
    @pl.when(pl.program_id(1) == 0)
    def _():
        sum_ref[...] = jnp.zeros_like(sum_ref)
        ssq_ref[...] = jnp.zeros_like(ssq_ref)

    w = w_ref[...]
    s = None
    q = None
    for b in range(nb):
        x = x_ref[b].astype(jnp.bfloat16)
        y = jnp.dot(w, x, preferred_element_type=jnp.float32)
        sb = jnp.sum(y, axis=1, keepdims=True)
        qb = jnp.sum(y * y, axis=1, keepdims=True)
        s = sb if s is None else s + sb
        q = qb if q is None else q + qb
    sum_ref[0] += s
    ssq_ref[0] += q


def _apply_kernel(x_ref, w_ref, psum_ref, pssq_ref, g_ref, b_ref, o_ref,
                  *, nb, r):
    mean = jnp.sum(psum_ref[...], axis=0) / r
    var = jnp.maximum(jnp.sum(pssq_ref[...], axis=0) / r - mean * mean, 0.0)
    scale = g_ref[...] * jax.lax.rsqrt(var + _BN_EPS)
    shift = b_ref[...] - mean * scale
    w = (w_ref[...] * scale).astype(jnp.bfloat16)
    for b in range(nb):
        x32 = x_ref[b]
        y = jnp.dot(w, x32.astype(jnp.bfloat16),
                    preferred_element_type=jnp.float32)
        o_ref[b] = jnp.maximum(y + shift, 0.0) + x32


def kernel(x, conv_w, conv_b, bn_gamma, bn_beta):
    del conv_b
    N, C_in, L = x.shape
    C_out = conv_w.shape[0]
    w32 = conv_w[:, :, 0].astype(jnp.float32)

    p = 2 if N % 2 == 0 else 1
    nb0 = next(b for b in (8, 4, 2, 1) if N % (p * b) == 0)
    nb1 = next(b for b in (4, 2, 1) if N % b == 0)
    steps0 = N // (p * nb0)

    w16_spec = pl.BlockSpec((C_out, C_in), lambda *_: (0, 0))
    x_spec0 = pl.BlockSpec((nb0, C_in, L),
                           lambda i, j: (i * steps0 + j, 0, 0))
    stat_spec = pl.BlockSpec((1, C_out, 1), lambda i, j: (i, 0, 0))
    psum, pssq = pl.pallas_call(
        functools.partial(_stats_kernel, nb=nb0),
        out_shape=(jax.ShapeDtypeStruct((p, C_out, 1), jnp.float32),
                   jax.ShapeDtypeStruct((p, C_out, 1), jnp.float32)),
        grid=(p, steps0),
        in_specs=[x_spec0, w16_spec],
        out_specs=(stat_spec, stat_spec),
        compiler_params=pltpu.CompilerParams(
            dimension_semantics=("parallel", "arbitrary"),
            vmem_limit_bytes=_VMEM_LIMIT),
    )(x, w32.astype(jnp.bfloat16))

    x_spec1 = pl.BlockSpec((nb1, C_in, L), lambda n: (n, 0, 0))
    w32_spec = pl.BlockSpec((C_out, C_in), lambda n: (0, 0))
    part_spec = pl.BlockSpec((p, C_out, 1), lambda n: (0, 0, 0))
    vec_spec = pl.BlockSpec((C_out, 1), lambda n: (0, 0))
    out = pl.pallas_call(
        functools.partial(_apply_kernel, nb=nb1, r=float(N * L)),
        out_shape=jax.ShapeDtypeStruct((N, C_out, L), x.dtype),
        grid=(N // nb1,),
        in_specs=[x_spec1, w32_spec, part_spec, part_spec, vec_spec,
                  vec_spec],
        out_specs=pl.BlockSpec((nb1, C_out, L), lambda n: (n, 0, 0)),
        compiler_params=pltpu.CompilerParams(
            dimension_semantics=("parallel",),
            vmem_limit_bytes=_VMEM_LIMIT),
    )(x, w32, psum, pssq, bn_gamma.reshape(C_out, 1),
      bn_beta.reshape(C_out, 1))
    return out
